# Initial kernel scaffold; baseline (speedup 1.0000x reference)
import jax
import jax.numpy as jnp
from jax.experimental import pallas as pl


def kernel(x, skipx, up_w, ops_0, ops_1):
    raise NotImplementedError("write your pallas kernel here")


# speedup vs baseline: 11.3457x; 11.3457x over previous
import functools

import numpy as np

import jax
import jax.numpy as jnp
from jax.experimental import pallas as pl
from jax.experimental.pallas import tpu as pltpu

_EPS = 1e-5


def _elu(y):
    return jnp.where(y > 0.0, y, jnp.exp(jnp.minimum(y, 0.0)) - 1.0)


def _in_elu(acc):
    mean = jnp.mean(acc, axis=-1, keepdims=True)
    cen = acc - mean
    var = jnp.mean(cen * cen, axis=-1, keepdims=True)
    return _elu(cen * jax.lax.rsqrt(var + _EPS))


def _conv3x3(xb, w, width):
    cin, hw = xb.shape
    lane_w = jax.lax.broadcasted_iota(jnp.int32, (cin, hw), 1) % width
    zc = jnp.zeros((cin, 1), xb.dtype)
    xl = jnp.where(lane_w == 0, jnp.array(0, xb.dtype),
                   jnp.concatenate([zc, xb[:, :-1]], axis=1))
    xr = jnp.where(lane_w == width - 1, jnp.array(0, xb.dtype),
                   jnp.concatenate([xb[:, 1:], zc], axis=1))
    row = jnp.concatenate([xl, xb, xr], axis=0)
    zr = jnp.zeros((3 * cin, width), xb.dtype)
    row_up = jnp.concatenate([zr, row[:, :-width]], axis=1)
    row_dn = jnp.concatenate([row[:, width:], zr], axis=1)
    k = 3 * cin
    acc = jnp.dot(w[:, :k], row_up, preferred_element_type=jnp.float32)
    acc = acc + jnp.dot(w[:, k:2 * k], row, preferred_element_type=jnp.float32)
    acc = acc + jnp.dot(w[:, 2 * k:], row_dn, preferred_element_type=jnp.float32)
    return acc


def _fused_kernel(width, up_ref, skip_ref, w1_ref, w2_ref, w3_ref, o_ref):
    ub = up_ref[0].astype(jnp.bfloat16)
    h1 = _in_elu(_conv3x3(ub, w1_ref[...], width))
    xcat = jnp.concatenate([h1, skip_ref[0]], axis=0)
    h2 = _in_elu(_conv3x3(xcat.astype(jnp.bfloat16), w2_ref[...], width))
    h3 = _in_elu(_conv3x3(h2.astype(jnp.bfloat16), w3_ref[...], width))
    o_ref[0] = _elu(h3 + xcat)


def _upsample_mat(n):
    u = np.zeros((2 * n, n), np.float32)
    for i in range(n):
        u[2 * i, i] += 0.75
        u[2 * i, max(i - 1, 0)] += 0.25
        u[2 * i + 1, i] += 0.75
        u[2 * i + 1, min(i + 1, n - 1)] += 0.25
    return jnp.asarray(u)


def kernel(x, skipx, up_w, ops_0, ops_1):
    n, cin, h, w = x.shape
    _, skip_c, hh, ww = skipx.shape
    hw = hh * ww
    hid_c = up_w.shape[0]
    out_c = ops_0.shape[0]

    uh = _upsample_mat(h)
    uw = _upsample_mat(w)
    up = jnp.einsum("ih,nchw->nciw", uh, x)
    up = jnp.einsum("jw,nciw->ncij", uw, up)

    up2 = up.reshape(n, cin, hw)
    skip2 = skipx.reshape(n, skip_c, hw)
    wb1 = up_w.astype(jnp.bfloat16)
    wb2 = ops_0.astype(jnp.bfloat16)
    wb3 = ops_1.astype(jnp.bfloat16)

    out = pl.pallas_call(
        functools.partial(_fused_kernel, ww),
        out_shape=jax.ShapeDtypeStruct((n, out_c, hw), jnp.float32),
        grid=(n,),
        in_specs=[
            pl.BlockSpec((1, cin, hw), lambda i: (i, 0, 0)),
            pl.BlockSpec((1, skip_c, hw), lambda i: (i, 0, 0)),
            pl.BlockSpec((hid_c, 9 * cin), lambda i: (0, 0)),
            pl.BlockSpec((out_c, 9 * out_c), lambda i: (0, 0)),
            pl.BlockSpec((out_c, 9 * out_c), lambda i: (0, 0)),
        ],
        out_specs=pl.BlockSpec((1, out_c, hw), lambda i: (i, 0, 0)),
        compiler_params=pltpu.CompilerParams(
            dimension_semantics=("parallel",)),
    )(up2, skip2, wb1, wb2, wb3)
    return out.reshape(n, out_c, hh, ww)


# speedup vs baseline: 13.4556x; 1.1860x over previous
import functools

import numpy as np

import jax
import jax.numpy as jnp
from jax.experimental import pallas as pl
from jax.experimental.pallas import tpu as pltpu

_EPS = 1e-5


def _elu(y):
    return jnp.where(y > 0.0, y, jnp.exp(jnp.minimum(y, 0.0)) - 1.0)


def _in_elu(acc):
    mean = jnp.mean(acc, axis=-1, keepdims=True)
    cen = acc - mean
    var = jnp.mean(cen * cen, axis=-1, keepdims=True)
    return _elu(cen * jax.lax.rsqrt(var + _EPS))


def _conv3x3(xb, w, width):
    cin, hw = xb.shape
    lane_w = jax.lax.broadcasted_iota(jnp.int32, (cin, hw), 1) % width
    zc = jnp.zeros((cin, 1), xb.dtype)
    xl = jnp.where(lane_w == 0, jnp.array(0, xb.dtype),
                   jnp.concatenate([zc, xb[:, :-1]], axis=1))
    xr = jnp.where(lane_w == width - 1, jnp.array(0, xb.dtype),
                   jnp.concatenate([xb[:, 1:], zc], axis=1))
    row = jnp.concatenate([xl, xb, xr], axis=0)
    zr = jnp.zeros((3 * cin, width), xb.dtype)
    row_up = jnp.concatenate([zr, row[:, :-width]], axis=1)
    row_dn = jnp.concatenate([row[:, width:], zr], axis=1)
    k = 3 * cin
    acc = jnp.dot(w[:, :k], row_up, preferred_element_type=jnp.float32)
    acc = acc + jnp.dot(w[:, k:2 * k], row, preferred_element_type=jnp.float32)
    acc = acc + jnp.dot(w[:, 2 * k:], row_dn, preferred_element_type=jnp.float32)
    return acc


def _fused_kernel(width, x_ref, skip_ref, up_mat_ref, w1_ref, w2_ref, w3_ref,
                  o_ref):
    xb = x_ref[0].astype(jnp.bfloat16)
    up = jnp.dot(xb, up_mat_ref[...], preferred_element_type=jnp.float32)
    ub = up.astype(jnp.bfloat16)
    h1 = _in_elu(_conv3x3(ub, w1_ref[...], width))
    xcat = jnp.concatenate([h1, skip_ref[0]], axis=0)
    h2 = _in_elu(_conv3x3(xcat.astype(jnp.bfloat16), w2_ref[...], width))
    h3 = _in_elu(_conv3x3(h2.astype(jnp.bfloat16), w3_ref[...], width))
    o_ref[0] = _elu(h3 + xcat)


def _upsample_mat(n):
    u = np.zeros((2 * n, n), np.float32)
    for i in range(n):
        u[2 * i, i] += 0.75
        u[2 * i, max(i - 1, 0)] += 0.25
        u[2 * i + 1, i] += 0.75
        u[2 * i + 1, min(i + 1, n - 1)] += 0.25
    return u


def kernel(x, skipx, up_w, ops_0, ops_1):
    n, cin, h, w = x.shape
    _, skip_c, hh, ww = skipx.shape
    hw = hh * ww
    hid_c = up_w.shape[0]
    out_c = ops_0.shape[0]

    uh = _upsample_mat(h)
    uw = _upsample_mat(w)
    up_mat = jnp.asarray(np.kron(uh, uw).T.astype(np.float32)).astype(
        jnp.bfloat16)

    x2 = x.reshape(n, cin, h * w)
    skip2 = skipx.reshape(n, skip_c, hw)
    wb1 = up_w.astype(jnp.bfloat16)
    wb2 = ops_0.astype(jnp.bfloat16)
    wb3 = ops_1.astype(jnp.bfloat16)

    out = pl.pallas_call(
        functools.partial(_fused_kernel, ww),
        out_shape=jax.ShapeDtypeStruct((n, out_c, hw), jnp.float32),
        grid=(n,),
        in_specs=[
            pl.BlockSpec((1, cin, h * w), lambda i: (i, 0, 0)),
            pl.BlockSpec((1, skip_c, hw), lambda i: (i, 0, 0)),
            pl.BlockSpec((h * w, hw), lambda i: (0, 0)),
            pl.BlockSpec((hid_c, 9 * cin), lambda i: (0, 0)),
            pl.BlockSpec((out_c, 9 * out_c), lambda i: (0, 0)),
            pl.BlockSpec((out_c, 9 * out_c), lambda i: (0, 0)),
        ],
        out_specs=pl.BlockSpec((1, out_c, hw), lambda i: (i, 0, 0)),
        compiler_params=pltpu.CompilerParams(
            dimension_semantics=("parallel",)),
    )(x2, skip2, up_mat, wb1, wb2, wb3)
    return out.reshape(n, out_c, hh, ww)


# speedup vs baseline: 15.6105x; 1.1601x over previous
import functools

import numpy as np

import jax
import jax.numpy as jnp
from jax.experimental import pallas as pl
from jax.experimental.pallas import tpu as pltpu

_EPS = 1e-5


def _elu(y):
    return jnp.where(y > 0.0, y, jnp.exp(jnp.minimum(y, 0.0)) - 1.0)


def _in_elu(acc):
    mean = jnp.mean(acc, axis=-1, keepdims=True)
    msq = jnp.mean(acc * acc, axis=-1, keepdims=True)
    var = msq - mean * mean
    scale = jax.lax.rsqrt(var + _EPS)
    return _elu((acc - mean) * scale)


def _conv3x3(xb, wstack, width):
    cin, hw = xb.shape
    cout = wstack.shape[0] // 3
    lane_w = jax.lax.broadcasted_iota(jnp.int32, (cin, hw), 1) % width
    zc = jnp.zeros((cin, 1), xb.dtype)
    xl = jnp.where(lane_w == 0, jnp.array(0, xb.dtype),
                   jnp.concatenate([zc, xb[:, :-1]], axis=1))
    xr = jnp.where(lane_w == width - 1, jnp.array(0, xb.dtype),
                   jnp.concatenate([xb[:, 1:], zc], axis=1))
    row = jnp.concatenate([xl, xb, xr], axis=0)
    z = jnp.dot(wstack, row, preferred_element_type=jnp.float32)
    za, zb, zc2 = z[:cout], z[cout:2 * cout], z[2 * cout:]
    zr = jnp.zeros((cout, width), jnp.float32)
    return (zb + jnp.concatenate([zr, za[:, :-width]], axis=1)
            + jnp.concatenate([zc2[:, width:], zr], axis=1))


def _fused_kernel(width, hid_c, x_ref, skip_ref, up_mat_ref, w1_ref, w2_ref,
                  w3_ref, o_ref):
    xb = x_ref[0].astype(jnp.bfloat16)
    up = jnp.dot(xb, up_mat_ref[...], preferred_element_type=jnp.float32)
    ub = up.astype(jnp.bfloat16)
    h1 = _in_elu(_conv3x3(ub, w1_ref[...], width))
    skip = skip_ref[0]
    xcatb = jnp.concatenate(
        [h1.astype(jnp.bfloat16), skip.astype(jnp.bfloat16)], axis=0)
    h2 = _in_elu(_conv3x3(xcatb, w2_ref[...], width))
    h3 = _in_elu(_conv3x3(h2.astype(jnp.bfloat16), w3_ref[...], width))
    o_ref[0, :hid_c] = _elu(h3[:hid_c] + h1)
    o_ref[0, hid_c:] = _elu(h3[hid_c:] + skip)


def _upsample_mat(n):
    u = np.zeros((2 * n, n), np.float32)
    for i in range(n):
        u[2 * i, i] += 0.75
        u[2 * i, max(i - 1, 0)] += 0.25
        u[2 * i + 1, i] += 0.75
        u[2 * i + 1, min(i + 1, n - 1)] += 0.25
    return u


def _stack_rows(w):
    cout, k9 = w.shape
    cin3 = k9 // 3
    return jnp.transpose(w.reshape(cout, 3, cin3), (1, 0, 2)).reshape(
        3 * cout, cin3).astype(jnp.bfloat16)


def kernel(x, skipx, up_w, ops_0, ops_1):
    n, cin, h, w = x.shape
    _, skip_c, hh, ww = skipx.shape
    hw = hh * ww
    hid_c = up_w.shape[0]
    out_c = ops_0.shape[0]

    up_mat = jnp.asarray(np.kron(_upsample_mat(h), _upsample_mat(w)).T,
                         dtype=jnp.bfloat16)

    x2 = x.reshape(n, cin, h * w)
    skip2 = skipx.reshape(n, skip_c, hw)
    wb1 = _stack_rows(up_w)
    wb2 = _stack_rows(ops_0)
    wb3 = _stack_rows(ops_1)

    out = pl.pallas_call(
        functools.partial(_fused_kernel, ww, hid_c),
        out_shape=jax.ShapeDtypeStruct((n, out_c, hw), jnp.float32),
        grid=(n,),
        in_specs=[
            pl.BlockSpec((1, cin, h * w), lambda i: (i, 0, 0)),
            pl.BlockSpec((1, skip_c, hw), lambda i: (i, 0, 0)),
            pl.BlockSpec((h * w, hw), lambda i: (0, 0)),
            pl.BlockSpec((3 * hid_c, 3 * cin), lambda i: (0, 0)),
            pl.BlockSpec((3 * out_c, 3 * out_c), lambda i: (0, 0)),
            pl.BlockSpec((3 * out_c, 3 * out_c), lambda i: (0, 0)),
        ],
        out_specs=pl.BlockSpec((1, out_c, hw), lambda i: (i, 0, 0)),
        compiler_params=pltpu.CompilerParams(
            dimension_semantics=("arbitrary",)),
    )(x2, skip2, up_mat, wb1, wb2, wb3)
    return out.reshape(n, out_c, hh, ww)


# speedup vs baseline: 19.4378x; 1.2452x over previous
import functools

import numpy as np

import jax
import jax.numpy as jnp
from jax.experimental import pallas as pl
from jax.experimental.pallas import tpu as pltpu

_EPS = 1e-5


def _elu(y):
    return jnp.where(y > 0.0, y, jnp.exp(jnp.minimum(y, 0.0)) - 1.0)


def _in_elu(acc, out_dtype=jnp.float32):
    mean = jnp.mean(acc, axis=-1, keepdims=True)
    msq = jnp.mean(acc * acc, axis=-1, keepdims=True)
    var = msq - mean * mean
    scale = jax.lax.rsqrt(var + _EPS)
    bias = -mean * scale
    return _elu(acc * scale + bias).astype(out_dtype)


def _conv3x3(xb, wstack, width):
    cin, hw = xb.shape
    cout = wstack.shape[0] // 3
    lane_w = jax.lax.broadcasted_iota(jnp.int32, (cin, hw), 1) % width
    zc = jnp.zeros((cin, 1), xb.dtype)
    xl = jnp.where(lane_w == 0, jnp.array(0, xb.dtype),
                   jnp.concatenate([zc, xb[:, :-1]], axis=1))
    xr = jnp.where(lane_w == width - 1, jnp.array(0, xb.dtype),
                   jnp.concatenate([xb[:, 1:], zc], axis=1))
    row = jnp.concatenate([xl, xb, xr], axis=0)
    z = jnp.dot(wstack, row, preferred_element_type=jnp.float32)
    za, zb, zc2 = z[:cout], z[cout:2 * cout], z[2 * cout:]
    zr = jnp.zeros((cout, width), jnp.float32)
    return (zb + jnp.concatenate([zr, za[:, :-width]], axis=1)
            + jnp.concatenate([zc2[:, width:], zr], axis=1))


def _fused_kernel(width, hid_c, x_ref, skip_ref, up_mat_ref, w1_ref, w2_ref,
                  w3_ref, o_ref):
    cin = x_ref.shape[1]
    skip_c = skip_ref.shape[1]
    out_c, hh, ww = o_ref.shape[1:]
    hw = hh * ww
    for s in range(x_ref.shape[0]):
        xb = x_ref[s].reshape(cin, -1).astype(jnp.bfloat16)
        up = jnp.dot(xb, up_mat_ref[...], preferred_element_type=jnp.float32)
        ub = up.astype(jnp.bfloat16)
        h1 = _in_elu(_conv3x3(ub, w1_ref[...], width))
        skip = skip_ref[s].reshape(skip_c, hw)
        xcatb = jnp.concatenate(
            [h1.astype(jnp.bfloat16), skip.astype(jnp.bfloat16)], axis=0)
        h2 = _in_elu(_conv3x3(xcatb, w2_ref[...], width), jnp.bfloat16)
        h3 = _in_elu(_conv3x3(h2, w3_ref[...], width))
        o_ref[s, :hid_c] = _elu(h3[:hid_c] + h1).reshape(hid_c, hh, ww)
        o_ref[s, hid_c:] = _elu(h3[hid_c:] + skip).reshape(
            out_c - hid_c, hh, ww)


def _upsample_mat(n):
    u = np.zeros((2 * n, n), np.float32)
    for i in range(n):
        u[2 * i, i] += 0.75
        u[2 * i, max(i - 1, 0)] += 0.25
        u[2 * i + 1, i] += 0.75
        u[2 * i + 1, min(i + 1, n - 1)] += 0.25
    return u


def _stack_rows(w):
    cout, k9 = w.shape
    cin3 = k9 // 3
    return jnp.transpose(w.reshape(cout, 3, cin3), (1, 0, 2)).reshape(
        3 * cout, cin3).astype(jnp.bfloat16)


def kernel(x, skipx, up_w, ops_0, ops_1):
    n, cin, h, w = x.shape
    _, skip_c, hh, ww = skipx.shape
    hw = hh * ww
    hid_c = up_w.shape[0]
    out_c = ops_0.shape[0]

    up_mat = jnp.asarray(np.kron(_upsample_mat(h), _upsample_mat(w)).T,
                         dtype=jnp.bfloat16)

    wb1 = _stack_rows(up_w)
    wb2 = _stack_rows(ops_0)
    wb3 = _stack_rows(ops_1)

    bs = 2 if n % 2 == 0 else 1
    out = pl.pallas_call(
        functools.partial(_fused_kernel, ww, hid_c),
        out_shape=jax.ShapeDtypeStruct((n, out_c, hh, ww), jnp.float32),
        grid=(n // bs,),
        in_specs=[
            pl.BlockSpec((bs, cin, h, w), lambda i: (i, 0, 0, 0)),
            pl.BlockSpec((bs, skip_c, hh, ww), lambda i: (i, 0, 0, 0)),
            pl.BlockSpec((h * w, hw), lambda i: (0, 0)),
            pl.BlockSpec((3 * hid_c, 3 * cin), lambda i: (0, 0)),
            pl.BlockSpec((3 * out_c, 3 * out_c), lambda i: (0, 0)),
            pl.BlockSpec((3 * out_c, 3 * out_c), lambda i: (0, 0)),
        ],
        out_specs=pl.BlockSpec((bs, out_c, hh, ww), lambda i: (i, 0, 0, 0)),
        compiler_params=pltpu.CompilerParams(
            dimension_semantics=("arbitrary",)),
    )(x, skipx, up_mat, wb1, wb2, wb3)
    return out
